# baseline (speedup 1.0000x reference)
import dataclasses
import functools

import jax
import jax.numpy as jnp
from jax import lax
from jax.experimental import pallas as pl
from jax.experimental.pallas import tpu as pltpu
from jax.experimental.pallas import tpu_sc as plsc

HT_BINS = 128 * 128
SPHERE = 32768
NUM_VOTES = 524288
CHANNELS = 64

NUM_CORES = 2
NUM_SUBCORES = 16
NUM_TILES = NUM_CORES * NUM_SUBCORES
CPT = CHANNELS // NUM_TILES
LANES = 16

CHUNK = 8192
NCHUNK = NUM_VOTES // CHUNK
NBUF = 2


def _compiler_params():
    cp = pltpu.CompilerParams()
    if "needs_layout_passes" in pltpu.CompilerParams.__dataclass_fields__:
        cp = dataclasses.replace(cp, needs_layout_passes=False)
    return cp


def _sphere_votes(x_pack, idx_pack, w):
    mesh = plsc.VectorSubcoreMesh(
        core_axis_name="core", subcore_axis_name="subcore"
    )

    @functools.partial(
        pl.kernel,
        out_type=jax.ShapeDtypeStruct((CHANNELS * SPHERE,), jnp.float32),
        mesh=mesh,
        scratch_types=[
            pltpu.VMEM((HT_BINS,), jnp.int32),
            pltpu.VMEM((CPT * SPHERE,), jnp.float32),
            pltpu.VMEM((NBUF, CHUNK), jnp.int32),
            pltpu.VMEM((NBUF, CHUNK), jnp.float32),
            pltpu.SemaphoreType.DMA((5,)),
        ],
        compiler_params=_compiler_params(),
    )
    def run(x_hbm, idx_hbm, w_hbm, out_hbm, x_v, acc_v, i_v, w_v, sem):
        cid = lax.axis_index("core")
        sid = lax.axis_index("subcore")
        wid = sid * NUM_CORES + cid
        c0 = wid * CPT

        x_copy = pltpu.async_copy(
            x_hbm.at[pl.ds(wid * HT_BINS, HT_BINS)], x_v, sem.at[4]
        )
        pltpu.async_copy(idx_hbm.at[0], i_v.at[0], sem.at[0])
        pltpu.async_copy(w_hbm.at[0], w_v.at[0], sem.at[2])

        zero = jnp.zeros((LANES,), jnp.float32)

        @plsc.parallel_loop(0, CPT * SPHERE // LANES, unroll=8)
        def _(j):
            acc_v[pl.ds(j * LANES, LANES)] = zero

        x_copy.wait()

        ht_mask = jnp.full((LANES,), HT_BINS - 1, jnp.int32)

        def process(b):
            @plsc.parallel_loop(0, CHUNK // LANES, unroll=16)
            def _(j):
                off = j * LANES
                iv = i_v[b, pl.ds(off, LANES)]
                ht16 = iv & ht_mask
                sph16 = lax.shift_right_logical(iv, 14)
                w16 = w_v[b, pl.ds(off, LANES)]
                gp = plsc.load_gather(x_v, [ht16])
                g0, g1 = plsc.unpack(
                    plsc.bitcast(gp, jnp.bfloat16),
                    format=plsc.PackFormat.INTERLEAVED,
                )
                plsc.addupdate_scatter(acc_v, [sph16], g0 * w16)
                plsc.addupdate_scatter(acc_v, [sph16 + SPHERE], g1 * w16)

        @pl.loop(0, NCHUNK, step=NBUF)
        def _(k):
            for b in range(NBUF):
                cur = k + b
                nxt = cur + 1

                @pl.when(nxt < NCHUNK)
                def _():
                    pltpu.async_copy(
                        idx_hbm.at[nxt], i_v.at[1 - b], sem.at[1 - b]
                    )
                    pltpu.async_copy(
                        w_hbm.at[nxt], w_v.at[1 - b], sem.at[2 + (1 - b)]
                    )

                pltpu.make_async_copy(
                    idx_hbm.at[cur], i_v.at[b], sem.at[b]
                ).wait()
                pltpu.make_async_copy(
                    w_hbm.at[cur], w_v.at[b], sem.at[2 + b]
                ).wait()
                process(b)

        pltpu.sync_copy(acc_v, out_hbm.at[pl.ds(c0 * SPHERE, CPT * SPHERE)])

    return run(x_pack, idx_pack, w)


def kernel(x, vote_ht, vote_sphere, vote_weight):
    batch, channel, h, w = x.shape
    xb = x.reshape(NUM_TILES, CPT, h * w).astype(jnp.bfloat16)
    xu = lax.bitcast_convert_type(xb, jnp.uint16).astype(jnp.uint32)
    x_pack = lax.bitcast_convert_type(
        xu[:, 0] | (xu[:, 1] << 16), jnp.int32
    ).reshape(NUM_TILES * HT_BINS)
    idx_pack = ((vote_sphere << 14) | vote_ht).reshape(NCHUNK, CHUNK)
    out = _sphere_votes(x_pack, idx_pack, vote_weight.reshape(NCHUNK, CHUNK))
    return out.reshape(batch, channel, SPHERE)


# speedup vs baseline: 1.0683x; 1.0683x over previous
import dataclasses
import functools

import jax
import jax.numpy as jnp
from jax import lax
from jax.experimental import pallas as pl
from jax.experimental.pallas import tpu as pltpu
from jax.experimental.pallas import tpu_sc as plsc

HT_BINS = 128 * 128
SPHERE = 32768
NUM_VOTES = 524288
CHANNELS = 64

NUM_CORES = 2
NUM_SUBCORES = 16
NUM_TILES = NUM_CORES * NUM_SUBCORES
CPT = CHANNELS // NUM_TILES
LANES = 16

CHUNK = 8192
NCHUNK = NUM_VOTES // CHUNK
NBUF = 2


def _compiler_params():
    cp = pltpu.CompilerParams()
    if "needs_layout_passes" in pltpu.CompilerParams.__dataclass_fields__:
        cp = dataclasses.replace(cp, needs_layout_passes=False)
    return cp


def _sphere_votes(x_pack, idx_pack, w):
    mesh = plsc.VectorSubcoreMesh(
        core_axis_name="core", subcore_axis_name="subcore"
    )

    @functools.partial(
        pl.kernel,
        out_type=jax.ShapeDtypeStruct((CHANNELS * SPHERE,), jnp.float32),
        mesh=mesh,
        scratch_types=[
            pltpu.VMEM((HT_BINS,), jnp.int32),
            pltpu.VMEM((CPT * SPHERE,), jnp.float32),
            pltpu.VMEM((NBUF, CHUNK), jnp.int32),
            pltpu.VMEM((NBUF, CHUNK), jnp.float32),
            pltpu.SemaphoreType.DMA((5,)),
        ],
        compiler_params=_compiler_params(),
    )
    def run(x_hbm, idx_hbm, w_hbm, out_hbm, x_v, acc_v, i_v, w_v, sem):
        cid = lax.axis_index("core")
        sid = lax.axis_index("subcore")
        wid = sid * NUM_CORES + cid
        c0 = wid * CPT

        x_copy = pltpu.async_copy(
            x_hbm.at[pl.ds(wid * HT_BINS, HT_BINS)], x_v, sem.at[4]
        )
        pltpu.async_copy(idx_hbm.at[0], i_v.at[0], sem.at[0])
        pltpu.async_copy(w_hbm.at[0], w_v.at[0], sem.at[2])

        zero = jnp.zeros((LANES,), jnp.float32)

        @plsc.parallel_loop(0, CPT * SPHERE // LANES, unroll=8)
        def _(j):
            acc_v[pl.ds(j * LANES, LANES)] = zero

        x_copy.wait()

        ht_mask = jnp.full((LANES,), HT_BINS - 1, jnp.int32)

        def process(b):
            @plsc.parallel_loop(0, CHUNK // LANES, unroll=8)
            def _(j):
                off = j * LANES
                iv = i_v[b, pl.ds(off, LANES)]
                ht16 = iv & ht_mask
                sph16 = lax.shift_right_logical(iv, 14)
                w16 = w_v[b, pl.ds(off, LANES)]
                gp = plsc.load_gather(x_v, [ht16])
                g0, g1 = plsc.unpack(
                    plsc.bitcast(gp, jnp.bfloat16),
                    format=plsc.PackFormat.INTERLEAVED,
                )
                plsc.addupdate_scatter(acc_v, [sph16], g0 * w16)
                plsc.addupdate_scatter(acc_v, [sph16 + SPHERE], g1 * w16)

        @pl.loop(0, NCHUNK, step=NBUF)
        def _(k):
            for b in range(NBUF):
                cur = k + b
                nxt = cur + 1

                @pl.when(nxt < NCHUNK)
                def _():
                    pltpu.async_copy(
                        idx_hbm.at[nxt], i_v.at[1 - b], sem.at[1 - b]
                    )
                    pltpu.async_copy(
                        w_hbm.at[nxt], w_v.at[1 - b], sem.at[2 + (1 - b)]
                    )

                pltpu.make_async_copy(
                    idx_hbm.at[cur], i_v.at[b], sem.at[b]
                ).wait()
                pltpu.make_async_copy(
                    w_hbm.at[cur], w_v.at[b], sem.at[2 + b]
                ).wait()
                process(b)

        pltpu.sync_copy(acc_v, out_hbm.at[pl.ds(c0 * SPHERE, CPT * SPHERE)])

    return run(x_pack, idx_pack, w)


def kernel(x, vote_ht, vote_sphere, vote_weight):
    batch, channel, h, w = x.shape
    xb = x.reshape(NUM_TILES, CPT, h * w).astype(jnp.bfloat16)
    xu = lax.bitcast_convert_type(xb, jnp.uint16).astype(jnp.uint32)
    x_pack = lax.bitcast_convert_type(
        xu[:, 0] | (xu[:, 1] << 16), jnp.int32
    ).reshape(NUM_TILES * HT_BINS)
    idx_pack = ((vote_sphere << 14) | vote_ht).reshape(NCHUNK, CHUNK)
    out = _sphere_votes(x_pack, idx_pack, vote_weight.reshape(NCHUNK, CHUNK))
    return out.reshape(batch, channel, SPHERE)


# speedup vs baseline: 1.2406x; 1.1613x over previous
import dataclasses
import functools

import jax
import jax.numpy as jnp
from jax import lax
from jax.experimental import pallas as pl
from jax.experimental.pallas import tpu as pltpu
from jax.experimental.pallas import tpu_sc as plsc

HT_BINS = 128 * 128
SPHERE = 32768
NUM_VOTES = 524288
CHANNELS = 64

NUM_CORES = 2
NUM_SUBCORES = 16
NUM_TILES = NUM_CORES * NUM_SUBCORES
CPT = CHANNELS // NUM_TILES
LANES = 16

CHUNK = 4096
NCHUNK = NUM_VOTES // CHUNK
NBUF = 4
assert NCHUNK % NBUF == 0


def _compiler_params():
    cp = pltpu.CompilerParams()
    if "needs_layout_passes" in pltpu.CompilerParams.__dataclass_fields__:
        cp = dataclasses.replace(cp, needs_layout_passes=False)
    return cp


def _sphere_votes(x_pack, votes_packed):
    mesh = plsc.VectorSubcoreMesh(
        core_axis_name="core", subcore_axis_name="subcore"
    )

    @functools.partial(
        pl.kernel,
        out_type=jax.ShapeDtypeStruct((CHANNELS * SPHERE,), jnp.float32),
        mesh=mesh,
        scratch_types=[
            pltpu.VMEM((HT_BINS,), jnp.int32),
            pltpu.VMEM((CPT * SPHERE,), jnp.float32),
            pltpu.VMEM((NBUF, 2, CHUNK), jnp.int32),
            pltpu.SemaphoreType.DMA((NBUF + 1,)),
        ],
        compiler_params=_compiler_params(),
    )
    def run(x_hbm, votes_hbm, out_hbm, x_v, acc_v, v_v, sem):
        cid = lax.axis_index("core")
        sid = lax.axis_index("subcore")
        wid = sid * NUM_CORES + cid
        c0 = wid * CPT

        x_copy = pltpu.async_copy(
            x_hbm.at[pl.ds(wid * HT_BINS, HT_BINS)], x_v, sem.at[NBUF]
        )
        for s in range(NBUF - 1):
            pltpu.async_copy(votes_hbm.at[s], v_v.at[s], sem.at[s])

        zero = jnp.zeros((LANES,), jnp.float32)

        @plsc.parallel_loop(0, CPT * SPHERE // LANES, unroll=8)
        def _(j):
            acc_v[pl.ds(j * LANES, LANES)] = zero

        x_copy.wait()

        ht_mask = jnp.full((LANES,), HT_BINS - 1, jnp.int32)

        def process(b):
            @plsc.parallel_loop(0, CHUNK // LANES, unroll=8)
            def _(j):
                off = j * LANES
                iv = v_v[b, 0, pl.ds(off, LANES)]
                ht16 = iv & ht_mask
                sph16 = lax.shift_right_logical(iv, 14)
                w16 = plsc.bitcast(v_v[b, 1, pl.ds(off, LANES)], jnp.float32)
                gp = plsc.load_gather(x_v, [ht16])
                g0, g1 = plsc.unpack(
                    plsc.bitcast(gp, jnp.bfloat16),
                    format=plsc.PackFormat.INTERLEAVED,
                )
                plsc.addupdate_scatter(acc_v, [sph16], g0 * w16)
                plsc.addupdate_scatter(acc_v, [sph16 + SPHERE], g1 * w16)

        @pl.loop(0, NCHUNK, step=NBUF)
        def _(k):
            for b in range(NBUF):
                cur = k + b
                pre = cur + (NBUF - 1)
                ps = (b + NBUF - 1) % NBUF

                @pl.when(pre < NCHUNK)
                def _():
                    pltpu.async_copy(
                        votes_hbm.at[pre], v_v.at[ps], sem.at[ps]
                    )

                pltpu.make_async_copy(
                    votes_hbm.at[cur], v_v.at[b], sem.at[b]
                ).wait()
                process(b)

        pltpu.sync_copy(acc_v, out_hbm.at[pl.ds(c0 * SPHERE, CPT * SPHERE)])

    return run(x_pack, votes_packed)


def kernel(x, vote_ht, vote_sphere, vote_weight):
    batch, channel, h, w = x.shape
    xb = x.reshape(NUM_TILES, CPT, h * w).astype(jnp.bfloat16)
    xu = lax.bitcast_convert_type(xb, jnp.uint16).astype(jnp.uint32)
    x_pack = lax.bitcast_convert_type(
        xu[:, 0] | (xu[:, 1] << 16), jnp.int32
    ).reshape(NUM_TILES * HT_BINS)
    idx_pack = (vote_sphere << 14) | vote_ht
    votes_packed = jnp.stack(
        [
            idx_pack.reshape(NCHUNK, CHUNK),
            lax.bitcast_convert_type(vote_weight, jnp.int32).reshape(
                NCHUNK, CHUNK
            ),
        ],
        axis=1,
    )
    out = _sphere_votes(x_pack, votes_packed)
    return out.reshape(batch, channel, SPHERE)
